# baseline (speedup 1.0000x reference)
import functools
import math

import numpy as np
import jax
import jax.numpy as jnp
from jax import lax
from jax.experimental import pallas as pl
from jax.experimental.pallas import tpu as pltpu
from jax.experimental.pallas import tpu_sc as plsc

S2 = math.sqrt(2.0)
NWORKERS = 32
NB, NF, NT, ROW = 32, 10, 500, 16
NBF = NB * NF

OSIZE = (ROW * (ROW + 1)) // 2
K0 = [ROW * i - i * (i - 1) // 2 for i in range(ROW + 1)]
NVC = (NT + 15) // 16
TAIL = NT - 16 * (NVC - 1)

_IOTA = np.arange(16, dtype=np.int32)
_TMASK = (_IOTA < TAIL)

mesh = plsc.VectorSubcoreMesh(core_axis_name="c", subcore_axis_name="s")

NBF_SC = 160
NBF_TC = NBF - NBF_SC
SLABS_PER_W = NBF_SC // NWORKERS


@functools.partial(
    pl.kernel,
    mesh=mesh,
    out_type=jax.ShapeDtypeStruct((NBF_SC * OSIZE, 1, NT), jnp.float32),
    scratch_types=[
        pltpu.VMEM((32, NT), jnp.float32),
        pltpu.VMEM((32, NT), jnp.float32),
        pltpu.VMEM((OSIZE, NT), jnp.float32),
        pltpu.SemaphoreType.DMA,
        pltpu.SemaphoreType.DMA,
        pltpu.SemaphoreType.DMA,
        pltpu.SemaphoreType.DMA,
    ],
    compiler_params=pltpu.CompilerParams(
        needs_layout_passes=False, use_tc_tiling_on_sc=True),
)
def _run(x_hbm, out_hbm, in0, in1, obuf, si0, si1, sd1, sd2):
    wid = lax.axis_index("s") * 2 + lax.axis_index("c")
    bf0 = wid * SLABS_PER_W
    iota = lax.broadcasted_iota(jnp.int32, (16,), 0)
    tmask = iota < TAIL
    in_bufs = (in0, in1)
    isems = (si0, si1)

    def in_copy(bf, c, b):
        return pltpu.make_async_copy(
            x_hbm.at[bf, pl.ds(32 * c, 32), :], in_bufs[b], isems[b])

    def d1_copy(bf):
        return pltpu.make_async_copy(
            obuf.at[pl.ds(0, 64), :],
            out_hbm.at[pl.ds(bf * OSIZE, 64), 0, :], sd1)

    def d2_copy(bf):
        return pltpu.make_async_copy(
            obuf.at[pl.ds(64, 72), :],
            out_hbm.at[pl.ds(bf * OSIZE + 64, 72), 0, :], sd2)

    def compute(c, b):
        ib = in_bufs[b]
        for di in range(2):
            i = 2 * c + di
            rloc0 = di * ROW + i
            k0 = K0[i]

            def row_body(jj, carry, rloc0=rloc0, k0=k0):
                r = rloc0 + jj
                k = k0 + jj
                scale = jnp.where(jj == 0, 1.0, S2)
                for v in range(NVC - 1):
                    vals = ib[r, pl.ds(16 * v, 16)]
                    obuf[k, pl.ds(16 * v, 16)] = jnp.abs(vals) * scale
                rvec = jnp.full((16,), r, jnp.int32)
                kvec = jnp.full((16,), k, jnp.int32)
                col = iota + 16 * (NVC - 1)
                vals = plsc.load_gather(ib, [rvec, col], mask=tmask)
                plsc.store_scatter(
                    obuf, [kvec, col], jnp.abs(vals) * scale, mask=tmask)
                return carry

            lax.fori_loop(0, ROW - i, row_body, 0)

    def bf_body(t, carry):
        bf = bf0 + t
        for c in range(8):
            b = c % 2
            in_copy(bf, c, b).wait()
            if c < 7:
                in_copy(bf, c + 1, 1 - b).start()
            else:
                @pl.when(t < SLABS_PER_W - 1)
                def _():
                    in_copy(bf + 1, 0, 1 - b).start()

            if c == 0:
                @pl.when(t > 0)
                def _(bf=bf):
                    d1_copy(bf - 1).wait()
            if c == 2:
                @pl.when(t > 0)
                def _(bf=bf):
                    d2_copy(bf - 1).wait()

            compute(c, b)
            if c == 2:
                d1_copy(bf).start()
        d2_copy(bf).start()
        return carry

    in_copy(bf0, 0, 0).start()
    lax.fori_loop(0, SLABS_PER_W, bf_body, 0)
    d1_copy(bf0 + SLABS_PER_W - 1).wait()
    d2_copy(bf0 + SLABS_PER_W - 1).wait()


_SEL = np.zeros((OSIZE, ROW * ROW), np.float32)
for _i in range(ROW):
    for _j in range(_i, ROW):
        _SEL[K0[_i] + (_j - _i), ROW * _i + _j] = 1.0 if _i == _j else S2


def _tc_body(s_ref, x_ref, o_ref):
    y = lax.dot_general(
        s_ref[...], x_ref[0],
        dimension_numbers=(((1,), (0,)), ((), ())),
        preferred_element_type=jnp.float32,
        precision=lax.Precision.HIGHEST)
    o_ref[...] = jnp.abs(y)[:, None, :]


_tc_run = pl.pallas_call(
    _tc_body,
    grid=(NBF_TC,),
    in_specs=[
        pl.BlockSpec((OSIZE, ROW * ROW), lambda i: (0, 0)),
        pl.BlockSpec((1, ROW * ROW, NT), lambda i: (i, 0, 0)),
    ],
    out_specs=pl.BlockSpec((OSIZE, 1, NT), lambda i: (i, 0, 0)),
    out_shape=jax.ShapeDtypeStruct((NBF_TC * OSIZE, 1, NT), jnp.float32),
)


def kernel(input_st):
    xv = input_st.transpose(0, 1, 3, 4, 2).reshape(NBF, ROW * ROW, NT)
    out_sc = _run(xv[:NBF_SC])
    out_tc = _tc_run(jnp.asarray(_SEL), xv[NBF_SC:])
    out = jnp.concatenate([out_sc, out_tc], axis=0)
    return out.reshape(NB, NF, OSIZE, 1, NT).transpose(0, 1, 4, 2, 3)


# speedup vs baseline: 2.7905x; 2.7905x over previous
import functools
import math

import numpy as np
import jax
import jax.numpy as jnp
from jax import lax
from jax.experimental import pallas as pl
from jax.experimental.pallas import tpu as pltpu
from jax.experimental.pallas import tpu_sc as plsc

S2 = math.sqrt(2.0)
NWORKERS = 32
NB, NF, NT, ROW = 32, 10, 500, 16
NBF = NB * NF

OSIZE = (ROW * (ROW + 1)) // 2
K0 = [ROW * i - i * (i - 1) // 2 for i in range(ROW + 1)]
NVC = (NT + 15) // 16
TAIL = NT - 16 * (NVC - 1)

_IOTA = np.arange(16, dtype=np.int32)
_TMASK = (_IOTA < TAIL)

mesh = plsc.VectorSubcoreMesh(core_axis_name="c", subcore_axis_name="s")

SLABS_PER_W = NBF // NWORKERS


@functools.partial(
    pl.kernel,
    mesh=mesh,
    out_type=jax.ShapeDtypeStruct((NBF * OSIZE, 1, NT), jnp.float32),
    scratch_types=[
        pltpu.VMEM((32, NT), jnp.float32),
        pltpu.VMEM((32, NT), jnp.float32),
        pltpu.VMEM((OSIZE, NT), jnp.float32),
        pltpu.SemaphoreType.DMA,
        pltpu.SemaphoreType.DMA,
        pltpu.SemaphoreType.DMA,
        pltpu.SemaphoreType.DMA,
    ],
    compiler_params=pltpu.CompilerParams(
        needs_layout_passes=False, use_tc_tiling_on_sc=True),
)
def _run(x_hbm, out_hbm, in0, in1, obuf, si0, si1, sd1, sd2):
    wid = lax.axis_index("s") * 2 + lax.axis_index("c")
    bf0 = wid * SLABS_PER_W
    iota = lax.broadcasted_iota(jnp.int32, (16,), 0)
    tmask = iota < TAIL
    in_bufs = (in0, in1)
    isems = (si0, si1)

    def in_copy(bf, c, b):
        return pltpu.make_async_copy(
            x_hbm.at[bf, pl.ds(32 * c, 32), :], in_bufs[b], isems[b])

    def d1_copy(bf):
        return pltpu.make_async_copy(
            obuf.at[pl.ds(0, 64), :],
            out_hbm.at[pl.ds(bf * OSIZE, 64), 0, :], sd1)

    def d2_copy(bf):
        return pltpu.make_async_copy(
            obuf.at[pl.ds(64, 72), :],
            out_hbm.at[pl.ds(bf * OSIZE + 64, 72), 0, :], sd2)

    def compute(c, b):
        ib = in_bufs[b]
        for di in range(2):
            i = 2 * c + di
            rloc0 = di * ROW + i
            k0 = K0[i]

            def row_body(jj, carry, rloc0=rloc0, k0=k0):
                r = rloc0 + jj
                k = k0 + jj
                scale = jnp.where(jj == 0, 1.0, S2)
                for v in range(NVC - 1):
                    vals = ib[r, pl.ds(16 * v, 16)]
                    obuf[k, pl.ds(16 * v, 16)] = jnp.abs(vals) * scale
                rvec = jnp.full((16,), r, jnp.int32)
                kvec = jnp.full((16,), k, jnp.int32)
                col = iota + 16 * (NVC - 1)
                vals = plsc.load_gather(ib, [rvec, col], mask=tmask)
                plsc.store_scatter(
                    obuf, [kvec, col], jnp.abs(vals) * scale, mask=tmask)
                return carry

            lax.fori_loop(0, ROW - i, row_body, 0)

    def bf_body(t, carry):
        bf = bf0 + t
        for c in range(8):
            b = c % 2
            in_copy(bf, c, b).wait()
            if c < 7:
                in_copy(bf, c + 1, 1 - b).start()
            else:
                @pl.when(t < SLABS_PER_W - 1)
                def _():
                    in_copy(bf + 1, 0, 1 - b).start()

            if c == 0:
                @pl.when(t > 0)
                def _(bf=bf):
                    d1_copy(bf - 1).wait()
            if c == 2:
                @pl.when(t > 0)
                def _(bf=bf):
                    d2_copy(bf - 1).wait()

            compute(c, b)
            if c == 2:
                d1_copy(bf).start()
        d2_copy(bf).start()
        return carry

    in_copy(bf0, 0, 0).start()
    lax.fori_loop(0, SLABS_PER_W, bf_body, 0)
    d1_copy(bf0 + SLABS_PER_W - 1).wait()
    d2_copy(bf0 + SLABS_PER_W - 1).wait()


def kernel(input_st):
    xv = input_st.transpose(0, 1, 3, 4, 2).reshape(NBF, ROW * ROW, NT)
    out = _run(xv)
    return out.reshape(NB, NF, OSIZE, 1, NT).transpose(0, 1, 4, 2, 3)


# speedup vs baseline: 2.7911x; 1.0002x over previous
import functools
import math

import numpy as np
import jax
import jax.numpy as jnp
from jax import lax
from jax.experimental import pallas as pl
from jax.experimental.pallas import tpu as pltpu
from jax.experimental.pallas import tpu_sc as plsc

S2 = math.sqrt(2.0)
NWORKERS = 32
NB, NF, NT, ROW = 32, 10, 500, 16
NBF = NB * NF

OSIZE = (ROW * (ROW + 1)) // 2
K0 = [ROW * i - i * (i - 1) // 2 for i in range(ROW + 1)]
NVC = (NT + 15) // 16
TAIL = NT - 16 * (NVC - 1)

_IOTA = np.arange(16, dtype=np.int32)
_TMASK = (_IOTA < TAIL)

mesh = plsc.VectorSubcoreMesh(core_axis_name="c", subcore_axis_name="s")

SLABS_PER_W = NBF // NWORKERS


@functools.partial(
    pl.kernel,
    mesh=mesh,
    out_type=jax.ShapeDtypeStruct((NBF * OSIZE, 1, NT), jnp.float32),
    scratch_types=[
        pltpu.VMEM((32, NT), jnp.float32),
        pltpu.VMEM((32, NT), jnp.float32),
        pltpu.VMEM((OSIZE, NT), jnp.float32),
        pltpu.SemaphoreType.DMA,
        pltpu.SemaphoreType.DMA,
        pltpu.SemaphoreType.DMA,
        pltpu.SemaphoreType.DMA,
    ],
    compiler_params=pltpu.CompilerParams(
        needs_layout_passes=False, use_tc_tiling_on_sc=True),
)
def _run(x_hbm, out_hbm, in0, in1, obuf, si0, si1, sd1, sd2):
    wid = lax.axis_index("s") * 2 + lax.axis_index("c")
    bf0 = wid * SLABS_PER_W
    iota = lax.broadcasted_iota(jnp.int32, (16,), 0)
    tmask = iota < TAIL
    in_bufs = (in0, in1)
    isems = (si0, si1)

    def in_copy(bf, c, b):
        return pltpu.make_async_copy(
            x_hbm.at[bf, pl.ds(32 * c, 32), :], in_bufs[b], isems[b])

    def d1_copy(bf):
        return pltpu.make_async_copy(
            obuf.at[pl.ds(0, 64), :],
            out_hbm.at[pl.ds(bf * OSIZE, 64), 0, :], sd1)

    def d2_copy(bf):
        return pltpu.make_async_copy(
            obuf.at[pl.ds(64, 72), :],
            out_hbm.at[pl.ds(bf * OSIZE + 64, 72), 0, :], sd2)

    def compute(c, b):
        ib = in_bufs[b]
        for di in range(2):
            i = 2 * c + di
            rloc0 = di * ROW + i
            k0 = K0[i]

            def row_body(jj, carry, rloc0=rloc0, k0=k0, diag=False):
                r = rloc0 + jj
                k = k0 + jj
                for v in range(NVC - 1):
                    vals = jnp.abs(ib[r, pl.ds(16 * v, 16)])
                    obuf[k, pl.ds(16 * v, 16)] = vals if diag else vals * S2
                rvec = jnp.full((16,), r, jnp.int32)
                kvec = jnp.full((16,), k, jnp.int32)
                col = iota + 16 * (NVC - 1)
                vals = jnp.abs(plsc.load_gather(ib, [rvec, col], mask=tmask))
                plsc.store_scatter(
                    obuf, [kvec, col], vals if diag else vals * S2,
                    mask=tmask)
                return carry

            row_body(0, 0, diag=True)
            lax.fori_loop(1, ROW - i, row_body, 0)

    def bf_body(t, carry):
        bf = bf0 + t
        for c in range(8):
            b = c % 2
            in_copy(bf, c, b).wait()
            if c < 7:
                in_copy(bf, c + 1, 1 - b).start()
            else:
                @pl.when(t < SLABS_PER_W - 1)
                def _():
                    in_copy(bf + 1, 0, 1 - b).start()

            if c == 0:
                @pl.when(t > 0)
                def _(bf=bf):
                    d1_copy(bf - 1).wait()
            if c == 2:
                @pl.when(t > 0)
                def _(bf=bf):
                    d2_copy(bf - 1).wait()

            compute(c, b)
            if c == 2:
                d1_copy(bf).start()
        d2_copy(bf).start()
        return carry

    in_copy(bf0, 0, 0).start()
    lax.fori_loop(0, SLABS_PER_W, bf_body, 0)
    d1_copy(bf0 + SLABS_PER_W - 1).wait()
    d2_copy(bf0 + SLABS_PER_W - 1).wait()


def kernel(input_st):
    xv = input_st.transpose(0, 1, 3, 4, 2).reshape(NBF, ROW * ROW, NT)
    out = _run(xv)
    return out.reshape(NB, NF, OSIZE, 1, NT).transpose(0, 1, 4, 2, 3)
